# baseline (speedup 1.0000x reference)
import functools

import jax
import jax.numpy as jnp
from jax import lax
from jax.experimental import pallas as pl
from jax.experimental.pallas import tpu as pltpu
from jax.experimental.pallas import tpu_sc as plsc

N = 10000
EM = 160000
EW = 32000
D = 128

NW = 32
CH = 128

GM = 2 * EM // CH
GW = 2 * EW // CH

MC = EM // CH
WC = EW // CH
MBUF = -(-MC // NW)
WBUF = -(-WC // NW)


def _mesh():
    return plsc.VectorSubcoreMesh(core_axis_name="c", subcore_axis_name="s")


_SC_PARAMS = pltpu.CompilerParams(use_tc_tiling_on_sc=False)


def _gather_body(ntot, nbuf, x_hbm, idx_hbm, out_hbm, idx_v,
                 r0, r1, r2, r3, g0, g1, g2, g3, t0, t1, t2, t3):
    wid = lax.axis_index("s") * 2 + lax.axis_index("c")
    lo = (wid * ntot) // NW
    cnt = ((wid + 1) * ntot) // NW - lo
    pltpu.sync_copy(idx_hbm.at[pl.ds(lo, nbuf)], idx_v)
    rows = (r0, r1, r2, r3)
    gsem = (g0, g1, g2, g3)
    ssem = (t0, t1, t2, t3)
    for b in range(2):
        pltpu.async_copy(x_hbm.at[idx_v.at[b]], rows[b], gsem[b])

    def round_body(r, carry):
        for jj in range(4):
            j = r * 4 + jj
            b2 = (jj + 2) % 4

            @pl.when(j + 2 < cnt)
            def _():
                @pl.when(j >= 2)
                def _():
                    pltpu.make_async_copy(
                        rows[b2], out_hbm.at[pl.ds(0, CH)], ssem[b2]).wait()
                pltpu.async_copy(x_hbm.at[idx_v.at[j + 2]], rows[b2], gsem[b2])

            @pl.when(j < cnt)
            def _():
                pltpu.make_async_copy(
                    x_hbm.at[idx_v.at[j]], rows[jj], gsem[jj]).wait()
                pltpu.async_copy(rows[jj], out_hbm.at[pl.ds((lo + j) * CH, CH)],
                                 ssem[jj])
        return carry

    lax.fori_loop(0, -(-nbuf // 4), round_body, 0)
    for b in range(4):
        pltpu.make_async_copy(rows[b], out_hbm.at[pl.ds(0, CH)], ssem[b]).wait()


def _make_gather(ntot):
  nbuf = -(-ntot // NW)

  @jax.jit
  def run(x, idx2d):
    return pl.kernel(
        functools.partial(_gather_body, ntot, nbuf),
        out_type=jax.ShapeDtypeStruct((ntot * CH, D), jnp.float32),
        mesh=_mesh(),
        compiler_params=_SC_PARAMS,
        scratch_types=[
            pltpu.VMEM((nbuf, CH), jnp.int32),
            pltpu.VMEM((CH, D), jnp.float32),
            pltpu.VMEM((CH, D), jnp.float32),
            pltpu.VMEM((CH, D), jnp.float32),
            pltpu.VMEM((CH, D), jnp.float32),
            pltpu.SemaphoreType.DMA,
            pltpu.SemaphoreType.DMA,
            pltpu.SemaphoreType.DMA,
            pltpu.SemaphoreType.DMA,
            pltpu.SemaphoreType.DMA,
            pltpu.SemaphoreType.DMA,
            pltpu.SemaphoreType.DMA,
            pltpu.SemaphoreType.DMA,
        ],
    )(x, idx2d)

  return run


_gather_mesh_h = _make_gather(GM // 2)
_gather_world = _make_gather(GW)


def _scatter_body(nchunks, nbuf, vals_hbm, idx_hbm, zt_hbm, out_hbm,
                  idx_v, r0, r1, s0, s1, shared):
    cid = lax.axis_index("c")
    sid = lax.axis_index("s")
    wid = sid * 2 + cid
    slab = N // 16
    pltpu.sync_copy(zt_hbm, shared.at[pl.ds(sid * slab, slab)])
    plsc.subcore_barrier()

    lo = (wid * nchunks) // NW
    cnt = ((wid + 1) * nchunks) // NW - lo
    pltpu.sync_copy(idx_hbm.at[pl.ds(lo, nbuf)], idx_v)
    rows = (r0, r1)
    sems = (s0, s1)
    for b in range(2):
        pltpu.async_copy(vals_hbm.at[pl.ds((lo + b) * CH, CH)], rows[b], sems[b])

    def round_body(r, carry):
        for b in range(2):
            j = r * 2 + b

            @pl.when(j < cnt)
            def _():
                pltpu.make_async_copy(
                    vals_hbm.at[pl.ds((lo + j) * CH, CH)], rows[b], sems[b]).wait()
                pltpu.sync_copy(rows[b], shared.at[idx_v.at[j]], add=True)

                @pl.when(j + 2 < cnt)
                def _():
                    pltpu.async_copy(
                        vals_hbm.at[pl.ds((lo + j + 2) * CH, CH)], rows[b], sems[b])
        return carry

    lax.fori_loop(0, (nbuf + 1) // 2, round_body, 0)
    plsc.subcore_barrier()
    pltpu.sync_copy(shared.at[pl.ds(sid * slab, slab)],
                    out_hbm.at[pl.ds(cid * N + sid * slab, slab)])


def _make_scatter(nchunks, nbuf):
    @jax.jit
    def run(vals, idx2d, zt):
        return pl.kernel(
            functools.partial(_scatter_body, nchunks, nbuf),
            out_type=jax.ShapeDtypeStruct((2 * N, D), jnp.float32),
            mesh=_mesh(),
            compiler_params=_SC_PARAMS,
            scratch_types=[
                pltpu.VMEM((nbuf, CH), jnp.int32),
                pltpu.VMEM((CH, D), jnp.float32),
                pltpu.VMEM((CH, D), jnp.float32),
                pltpu.SemaphoreType.DMA,
                pltpu.SemaphoreType.DMA,
                pltpu.VMEM_SHARED((N, D), jnp.float32),
            ],
        )(vals, idx2d, zt)

    return run


_scatter_mesh = _make_scatter(MC, MBUF)
_scatter_world = _make_scatter(WC, WBUF)


def _counts_body(nchunks, nbuf, ones_hbm, idx_hbm, zt_hbm, out_hbm,
                 idx_v, ones_v, shared):
    cid = lax.axis_index("c")
    sid = lax.axis_index("s")
    wid = sid * 2 + cid
    slab = N // 16
    pltpu.sync_copy(zt_hbm, shared.at[pl.ds(sid * slab, slab)])
    pltpu.sync_copy(ones_hbm, ones_v)
    plsc.subcore_barrier()

    lo = (wid * nchunks) // NW
    cnt = ((wid + 1) * nchunks) // NW - lo
    pltpu.sync_copy(idx_hbm.at[pl.ds(lo, nbuf)], idx_v)

    def body(j, carry):
        pltpu.sync_copy(ones_v, shared.at[idx_v.at[j]], add=True)
        return carry

    lax.fori_loop(0, cnt, body, 0)
    plsc.subcore_barrier()
    pltpu.sync_copy(shared.at[pl.ds(sid * slab, slab)],
                    out_hbm.at[pl.ds(cid * N + sid * slab, slab)])


def _make_counts(nchunks, nbuf):
    @jax.jit
    def run(ones_t, idx2d, zt):
        return pl.kernel(
            functools.partial(_counts_body, nchunks, nbuf),
            out_type=jax.ShapeDtypeStruct((2 * N, D), jnp.float32),
            mesh=_mesh(),
            compiler_params=_SC_PARAMS,
            scratch_types=[
                pltpu.VMEM((nbuf, CH), jnp.int32),
                pltpu.VMEM((CH, D), jnp.float32),
                pltpu.VMEM_SHARED((N, D), jnp.float32),
            ],
        )(ones_t, idx2d, zt)

    return run


_counts_mesh = _make_counts(MC, MBUF)
_counts_world = _make_counts(WC, WBUF)


def _mlp_ln(pre, W2, b2, W3, b3, g, beta):
    h = jax.nn.relu(pre)
    h = jax.nn.relu(jnp.dot(h, W2, preferred_element_type=jnp.float32) + b2)
    h = jnp.dot(h, W3, preferred_element_type=jnp.float32) + b3
    mu = jnp.mean(h, axis=-1, keepdims=True)
    hc = h - mu
    var = jnp.mean(hc * hc, axis=-1, keepdims=True)
    return hc * lax.rsqrt(var + 1e-5) * g + beta


def _edge_body(xs_ref, xr_ref, ea_ref, W1a, W1b, W1c, W2, W3,
               b1, b2, b3, g, beta, out_ref):
    xs = xs_ref[...]
    xr = xr_ref[...]
    ea = ea_ref[...]
    pre = (jnp.dot(xs, W1a[...], preferred_element_type=jnp.float32)
           + jnp.dot(xr, W1b[...], preferred_element_type=jnp.float32)
           + jnp.dot(ea, W1c[...], preferred_element_type=jnp.float32)
           + b1[...])
    out_ref[...] = ea + _mlp_ln(pre, W2[...], b2[...], W3[...], b3[...],
                                g[...], beta[...])


def _make_edge(nblocks, bs, off_s, off_r):
    wspec = pl.BlockSpec((D, D), lambda i: (0, 0))
    vspec = pl.BlockSpec((1, D), lambda i: (0, 0))

    @jax.jit
    def run(gat, ea, W1a, W1b, W1c, W2, W3, b1, b2, b3, g, beta):
        return pl.pallas_call(
            _edge_body,
            grid=(nblocks,),
            in_specs=[
                pl.BlockSpec((bs, D), lambda i: (i + off_s, 0)),
                pl.BlockSpec((bs, D), lambda i: (i + off_r, 0)),
                pl.BlockSpec((bs, D), lambda i: (i, 0)),
                wspec, wspec, wspec, wspec, wspec,
                vspec, vspec, vspec, vspec, vspec,
            ],
            out_specs=pl.BlockSpec((bs, D), lambda i: (i, 0)),
            out_shape=jax.ShapeDtypeStruct(ea.shape, jnp.float32),
        )(gat, gat, ea, W1a, W1b, W1c, W2, W3, b1, b2, b3, g, beta)

    return run


EB = 8000
_edge_world = _make_edge(EW // EB, EB, 0, EW // EB)

EMH = EM // 2


def _make_edge_half(h):
    nblocks = EMH // EB
    off = h * nblocks
    wspec = pl.BlockSpec((D, D), lambda i: (0, 0))
    vspec = pl.BlockSpec((1, D), lambda i: (0, 0))

    @jax.jit
    def run(gat, ea, W1a, W1b, W1c, W2, W3, b1, b2, b3, g, beta):
        return pl.pallas_call(
            _edge_body,
            grid=(nblocks,),
            in_specs=[
                pl.BlockSpec((EB, D), lambda i: (i, 0)),
                pl.BlockSpec((EB, D), lambda i: (i + nblocks, 0)),
                pl.BlockSpec((EB, D), lambda i: (i + off, 0)),
                wspec, wspec, wspec, wspec, wspec,
                vspec, vspec, vspec, vspec, vspec,
            ],
            out_specs=pl.BlockSpec((EB, D), lambda i: (i + off, 0)),
            out_shape=jax.ShapeDtypeStruct((EM, D), jnp.float32),
            input_output_aliases={2: 0},
        )(gat, gat, ea, W1a, W1b, W1c, W2, W3, b1, b2, b3, g, beta)

    return run


_edge_mesh_h0 = _make_edge_half(0)
_edge_mesh_h1 = _make_edge_half(1)


def _node_body(x_ref, sm0, sm1, sw0, sw1, cm0, cm1, cw0, cw1,
               W1a, W1b, W1c, W2, W3, b1, b2, b3, g, beta, out_ref):
    x = x_ref[...]
    aggm = (sm0[...] + sm1[...]) / jnp.maximum(cm0[...] + cm1[...], 1.0)
    aggw = (sw0[...] + sw1[...]) / jnp.maximum(cw0[...] + cw1[...], 1.0)
    pre = (jnp.dot(x, W1a[...], preferred_element_type=jnp.float32)
           + jnp.dot(aggm, W1b[...], preferred_element_type=jnp.float32)
           + jnp.dot(aggw, W1c[...], preferred_element_type=jnp.float32)
           + b1[...])
    out_ref[...] = x + _mlp_ln(pre, W2[...], b2[...], W3[...], b3[...],
                               g[...], beta[...])


NB_BLK = 2000


@jax.jit
def _node(x, summ, sumw, cntm, cntw, W1a, W1b, W1c, W2, W3, b1, b2, b3, g, beta):
    nblocks = N // NB_BLK
    off = N // NB_BLK
    wspec = pl.BlockSpec((D, D), lambda i: (0, 0))
    vspec = pl.BlockSpec((1, D), lambda i: (0, 0))
    blk = lambda: pl.BlockSpec((NB_BLK, D), lambda i: (i, 0))
    blk_off = lambda: pl.BlockSpec((NB_BLK, D), lambda i: (i + off, 0))
    return pl.pallas_call(
        _node_body,
        grid=(nblocks,),
        in_specs=[
            blk(),
            blk(), blk_off(),
            blk(), blk_off(),
            blk(), blk_off(),
            blk(), blk_off(),
            wspec, wspec, wspec, wspec, wspec,
            vspec, vspec, vspec, vspec, vspec,
        ],
        out_specs=pl.BlockSpec((NB_BLK, D), lambda i: (i, 0)),
        out_shape=jax.ShapeDtypeStruct((N, D), jnp.float32),
    )(x, summ, summ, sumw, sumw, cntm, cntm, cntw, cntw,
      W1a, W1b, W1c, W2, W3, b1, b2, b3, g, beta)


def _pad2d(idx, rows):
    return jnp.pad(idx, (0, rows * CH - idx.shape[0])).reshape(rows, CH)


def kernel(x, mesh_edge_index, mesh_edge_attr, world_edge_index, world_edge_attr,
           W1, b1, W2, b2, W3, b3, g, beta):
    steps = W1.shape[0]
    sm, rm = mesh_edge_index[0], mesh_edge_index[1]
    sw, rw = world_edge_index[0], world_edge_index[1]

    idx_m0 = _pad2d(jnp.concatenate([sm[:EMH], rm[:EMH]]), GM // 2)
    idx_m1 = _pad2d(jnp.concatenate([sm[EMH:], rm[EMH:]]), GM // 2)
    idx_gw = _pad2d(jnp.concatenate([sw, rw]), GW)
    rm2 = _pad2d(rm, MC + MBUF)
    rw2 = _pad2d(rw, WC + WBUF)
    zt = jnp.zeros((N // 16, D), jnp.float32)
    ones_t = jnp.ones((CH, D), jnp.float32)

    cntm = _counts_mesh(ones_t, rm2, zt)
    cntw = _counts_world(ones_t, rw2, zt)

    def wp(si, m):
        return (W1[si, m, :D], W1[si, m, D:2 * D], W1[si, m, 2 * D:],
                W2[si, m], W3[si, m],
                b1[si, m].reshape(1, D), b2[si, m].reshape(1, D),
                b3[si, m].reshape(1, D), g[si, m].reshape(1, D),
                beta[si, m].reshape(1, D))

    for si in range(steps):
        g0 = _gather_mesh_h(x, idx_m0)
        g1 = _gather_mesh_h(x, idx_m1)
        gat_w = _gather_world(x, idx_gw)
        mesh_edge_attr = _edge_mesh_h0(g0, mesh_edge_attr, *wp(si, 0))
        mesh_edge_attr = _edge_mesh_h1(g1, mesh_edge_attr, *wp(si, 0))
        summ = _scatter_mesh(mesh_edge_attr, rm2, zt)
        world_edge_attr = _edge_world(gat_w, world_edge_attr, *wp(si, 1))
        sumw = _scatter_world(world_edge_attr, rw2, zt)
        x = _node(x, summ, sumw, cntm, cntw, *wp(si, 2))

    return (x, mesh_edge_attr, world_edge_attr)


# speedup vs baseline: 1.0767x; 1.0767x over previous
import functools

import jax
import jax.numpy as jnp
from jax import lax
from jax.experimental import pallas as pl
from jax.experimental.pallas import tpu as pltpu
from jax.experimental.pallas import tpu_sc as plsc

N = 10000
EM = 160000
EW = 32000
D = 128

NW = 32
CH = 128

GM = 2 * EM // CH
GW = 2 * EW // CH

MC = EM // CH
WC = EW // CH
MBUF = -(-MC // NW)
WBUF = -(-WC // NW)


def _mesh():
    return plsc.VectorSubcoreMesh(core_axis_name="c", subcore_axis_name="s")


_SC_PARAMS = pltpu.CompilerParams(use_tc_tiling_on_sc=False)


def _gather_body(ntot, nbuf, x_hbm, idx_hbm, out_hbm, idx_v,
                 r0, r1, r2, r3, g0, g1, g2, g3, t0, t1, t2, t3):
    wid = lax.axis_index("s") * 2 + lax.axis_index("c")
    lo = (wid * ntot) // NW
    cnt = ((wid + 1) * ntot) // NW - lo
    pltpu.sync_copy(idx_hbm.at[pl.ds(lo, nbuf)], idx_v)
    rows = (r0, r1, r2, r3)
    gsem = (g0, g1, g2, g3)
    ssem = (t0, t1, t2, t3)
    for b in range(2):
        pltpu.async_copy(x_hbm.at[idx_v.at[b]], rows[b], gsem[b])

    def round_body(r, carry):
        for jj in range(4):
            j = r * 4 + jj
            b2 = (jj + 2) % 4

            @pl.when(j + 2 < cnt)
            def _():
                @pl.when(j >= 2)
                def _():
                    pltpu.make_async_copy(
                        rows[b2], out_hbm.at[pl.ds(0, CH)], ssem[b2]).wait()
                pltpu.async_copy(x_hbm.at[idx_v.at[j + 2]], rows[b2], gsem[b2])

            @pl.when(j < cnt)
            def _():
                pltpu.make_async_copy(
                    x_hbm.at[idx_v.at[j]], rows[jj], gsem[jj]).wait()
                pltpu.async_copy(rows[jj], out_hbm.at[pl.ds((lo + j) * CH, CH)],
                                 ssem[jj])
        return carry

    lax.fori_loop(0, -(-nbuf // 4), round_body, 0)
    for b in range(4):
        pltpu.make_async_copy(rows[b], out_hbm.at[pl.ds(0, CH)], ssem[b]).wait()


def _make_gather(ntot):
  nbuf = -(-ntot // NW)

  @jax.jit
  def run(x, idx2d):
    return pl.kernel(
        functools.partial(_gather_body, ntot, nbuf),
        out_type=jax.ShapeDtypeStruct((ntot * CH, D), jnp.float32),
        mesh=_mesh(),
        compiler_params=_SC_PARAMS,
        scratch_types=[
            pltpu.VMEM((nbuf, CH), jnp.int32),
            pltpu.VMEM((CH, D), jnp.float32),
            pltpu.VMEM((CH, D), jnp.float32),
            pltpu.VMEM((CH, D), jnp.float32),
            pltpu.VMEM((CH, D), jnp.float32),
            pltpu.SemaphoreType.DMA,
            pltpu.SemaphoreType.DMA,
            pltpu.SemaphoreType.DMA,
            pltpu.SemaphoreType.DMA,
            pltpu.SemaphoreType.DMA,
            pltpu.SemaphoreType.DMA,
            pltpu.SemaphoreType.DMA,
            pltpu.SemaphoreType.DMA,
        ],
    )(x, idx2d)

  return run


_gather_mesh_h = _make_gather(GM // 2)
_gather_world = _make_gather(GW)


def _scatter_body(nchunks, nbuf, vals_hbm, idx_hbm, zt_hbm, out_hbm,
                  idx_v, r0, r1, s0, s1, shared):
    cid = lax.axis_index("c")
    sid = lax.axis_index("s")
    wid = sid * 2 + cid
    slab = N // 16
    pltpu.sync_copy(zt_hbm, shared.at[pl.ds(sid * slab, slab)])
    plsc.subcore_barrier()

    lo = (wid * nchunks) // NW
    cnt = ((wid + 1) * nchunks) // NW - lo
    pltpu.sync_copy(idx_hbm.at[pl.ds(lo, nbuf)], idx_v)
    rows = (r0, r1)
    sems = (s0, s1)
    for b in range(2):
        pltpu.async_copy(vals_hbm.at[pl.ds((lo + b) * CH, CH)], rows[b], sems[b])

    def round_body(r, carry):
        for b in range(2):
            j = r * 2 + b

            @pl.when(j < cnt)
            def _():
                pltpu.make_async_copy(
                    vals_hbm.at[pl.ds((lo + j) * CH, CH)], rows[b], sems[b]).wait()
                pltpu.sync_copy(rows[b], shared.at[idx_v.at[j]], add=True)

                @pl.when(j + 2 < cnt)
                def _():
                    pltpu.async_copy(
                        vals_hbm.at[pl.ds((lo + j + 2) * CH, CH)], rows[b], sems[b])
        return carry

    lax.fori_loop(0, (nbuf + 1) // 2, round_body, 0)
    plsc.subcore_barrier()
    pltpu.sync_copy(shared.at[pl.ds(sid * slab, slab)],
                    out_hbm.at[pl.ds(cid * N + sid * slab, slab)])


def _make_scatter(nchunks, nbuf):
    @jax.jit
    def run(vals, idx2d, zt):
        return pl.kernel(
            functools.partial(_scatter_body, nchunks, nbuf),
            out_type=jax.ShapeDtypeStruct((2 * N, D), jnp.float32),
            mesh=_mesh(),
            compiler_params=_SC_PARAMS,
            scratch_types=[
                pltpu.VMEM((nbuf, CH), jnp.int32),
                pltpu.VMEM((CH, D), jnp.float32),
                pltpu.VMEM((CH, D), jnp.float32),
                pltpu.SemaphoreType.DMA,
                pltpu.SemaphoreType.DMA,
                pltpu.VMEM_SHARED((N, D), jnp.float32),
            ],
        )(vals, idx2d, zt)

    return run


_scatter_mesh = _make_scatter(MC, MBUF)
_scatter_world = _make_scatter(WC, WBUF)


def _counts_body(nchunks, nbuf, ones_hbm, idx_hbm, zt_hbm, out_hbm,
                 idx_v, ones_v, shared):
    cid = lax.axis_index("c")
    sid = lax.axis_index("s")
    wid = sid * 2 + cid
    slab = N // 16
    pltpu.sync_copy(zt_hbm, shared.at[pl.ds(sid * slab, slab)])
    pltpu.sync_copy(ones_hbm, ones_v)
    plsc.subcore_barrier()

    lo = (wid * nchunks) // NW
    cnt = ((wid + 1) * nchunks) // NW - lo
    pltpu.sync_copy(idx_hbm.at[pl.ds(lo, nbuf)], idx_v)

    def body(j, carry):
        pltpu.sync_copy(ones_v, shared.at[idx_v.at[j]], add=True)
        return carry

    lax.fori_loop(0, cnt, body, 0)
    plsc.subcore_barrier()
    pltpu.sync_copy(shared.at[pl.ds(sid * slab, slab)],
                    out_hbm.at[pl.ds(cid * N + sid * slab, slab)])


def _make_counts(nchunks, nbuf):
    @jax.jit
    def run(ones_t, idx2d, zt):
        return pl.kernel(
            functools.partial(_counts_body, nchunks, nbuf),
            out_type=jax.ShapeDtypeStruct((2 * N, D), jnp.float32),
            mesh=_mesh(),
            compiler_params=_SC_PARAMS,
            scratch_types=[
                pltpu.VMEM((nbuf, CH), jnp.int32),
                pltpu.VMEM((CH, D), jnp.float32),
                pltpu.VMEM_SHARED((N, D), jnp.float32),
            ],
        )(ones_t, idx2d, zt)

    return run


_counts_mesh = _make_counts(MC, MBUF)
_counts_world = _make_counts(WC, WBUF)


def _mlp_ln(pre, W2, b2, W3, b3, g, beta):
    h = jax.nn.relu(pre)
    h = jax.nn.relu(jnp.dot(h, W2, preferred_element_type=jnp.float32) + b2)
    h = jnp.dot(h, W3, preferred_element_type=jnp.float32) + b3
    mu = jnp.mean(h, axis=-1, keepdims=True)
    hc = h - mu
    var = jnp.mean(hc * hc, axis=-1, keepdims=True)
    return hc * lax.rsqrt(var + 1e-5) * g + beta


def _edge_body(xs_ref, xr_ref, ea_ref, W1a, W1b, W1c, W2, W3,
               b1, b2, b3, g, beta, out_ref):
    xs = xs_ref[...]
    xr = xr_ref[...]
    ea = ea_ref[...]
    pre = (jnp.dot(xs, W1a[...], preferred_element_type=jnp.float32)
           + jnp.dot(xr, W1b[...], preferred_element_type=jnp.float32)
           + jnp.dot(ea, W1c[...], preferred_element_type=jnp.float32)
           + b1[...])
    out_ref[...] = ea + _mlp_ln(pre, W2[...], b2[...], W3[...], b3[...],
                                g[...], beta[...])


def _make_edge(nblocks, bs, off_s, off_r):
    wspec = pl.BlockSpec((D, D), lambda i: (0, 0))
    vspec = pl.BlockSpec((1, D), lambda i: (0, 0))

    @jax.jit
    def run(gat, ea, W1a, W1b, W1c, W2, W3, b1, b2, b3, g, beta):
        return pl.pallas_call(
            _edge_body,
            grid=(nblocks,),
            in_specs=[
                pl.BlockSpec((bs, D), lambda i: (i + off_s, 0)),
                pl.BlockSpec((bs, D), lambda i: (i + off_r, 0)),
                pl.BlockSpec((bs, D), lambda i: (i, 0)),
                wspec, wspec, wspec, wspec, wspec,
                vspec, vspec, vspec, vspec, vspec,
            ],
            out_specs=pl.BlockSpec((bs, D), lambda i: (i, 0)),
            out_shape=jax.ShapeDtypeStruct(ea.shape, jnp.float32),
        )(gat, gat, ea, W1a, W1b, W1c, W2, W3, b1, b2, b3, g, beta)

    return run


EB = 8000
_edge_world = _make_edge(EW // EB, EB, 0, EW // EB)

EMH = EM // 2


def _edge_body_p(xs_ref, xr_ref, ea_ref, prev_ref, W1a, W1b, W1c, W2, W3,
                 b1, b2, b3, g, beta, out_ref):
    del prev_ref
    _edge_body(xs_ref, xr_ref, ea_ref, W1a, W1b, W1c, W2, W3,
               b1, b2, b3, g, beta, out_ref)


_NBH = EMH // EB


@jax.jit
def _edge_mesh_h0(gat, ea, W1a, W1b, W1c, W2, W3, b1, b2, b3, g, beta):
    wspec = pl.BlockSpec((D, D), lambda i: (0, 0))
    vspec = pl.BlockSpec((1, D), lambda i: (0, 0))
    return pl.pallas_call(
        _edge_body,
        grid=(_NBH,),
        in_specs=[
            pl.BlockSpec((EB, D), lambda i: (i, 0)),
            pl.BlockSpec((EB, D), lambda i: (i + _NBH, 0)),
            pl.BlockSpec((EB, D), lambda i: (i, 0)),
            wspec, wspec, wspec, wspec, wspec,
            vspec, vspec, vspec, vspec, vspec,
        ],
        out_specs=pl.BlockSpec((EB, D), lambda i: (i, 0)),
        out_shape=jax.ShapeDtypeStruct((EM, D), jnp.float32),
    )(gat, gat, ea, W1a, W1b, W1c, W2, W3, b1, b2, b3, g, beta)


@jax.jit
def _edge_mesh_h1(gat, ea, prev, W1a, W1b, W1c, W2, W3, b1, b2, b3, g, beta):
    wspec = pl.BlockSpec((D, D), lambda i: (0, 0))
    vspec = pl.BlockSpec((1, D), lambda i: (0, 0))
    return pl.pallas_call(
        _edge_body_p,
        grid=(_NBH,),
        in_specs=[
            pl.BlockSpec((EB, D), lambda i: (i, 0)),
            pl.BlockSpec((EB, D), lambda i: (i + _NBH, 0)),
            pl.BlockSpec((EB, D), lambda i: (i + _NBH, 0)),
            pl.BlockSpec(memory_space=pl.ANY),
            wspec, wspec, wspec, wspec, wspec,
            vspec, vspec, vspec, vspec, vspec,
        ],
        out_specs=pl.BlockSpec((EB, D), lambda i: (i + _NBH, 0)),
        out_shape=jax.ShapeDtypeStruct((EM, D), jnp.float32),
        input_output_aliases={3: 0},
    )(gat, gat, ea, prev, W1a, W1b, W1c, W2, W3, b1, b2, b3, g, beta)


def _node_body(x_ref, sm0, sm1, sw0, sw1, cm0, cm1, cw0, cw1,
               W1a, W1b, W1c, W2, W3, b1, b2, b3, g, beta, out_ref):
    x = x_ref[...]
    aggm = (sm0[...] + sm1[...]) / jnp.maximum(cm0[...] + cm1[...], 1.0)
    aggw = (sw0[...] + sw1[...]) / jnp.maximum(cw0[...] + cw1[...], 1.0)
    pre = (jnp.dot(x, W1a[...], preferred_element_type=jnp.float32)
           + jnp.dot(aggm, W1b[...], preferred_element_type=jnp.float32)
           + jnp.dot(aggw, W1c[...], preferred_element_type=jnp.float32)
           + b1[...])
    out_ref[...] = x + _mlp_ln(pre, W2[...], b2[...], W3[...], b3[...],
                               g[...], beta[...])


NB_BLK = 2000


@jax.jit
def _node(x, summ, sumw, cntm, cntw, W1a, W1b, W1c, W2, W3, b1, b2, b3, g, beta):
    nblocks = N // NB_BLK
    off = N // NB_BLK
    wspec = pl.BlockSpec((D, D), lambda i: (0, 0))
    vspec = pl.BlockSpec((1, D), lambda i: (0, 0))
    blk = lambda: pl.BlockSpec((NB_BLK, D), lambda i: (i, 0))
    blk_off = lambda: pl.BlockSpec((NB_BLK, D), lambda i: (i + off, 0))
    return pl.pallas_call(
        _node_body,
        grid=(nblocks,),
        in_specs=[
            blk(),
            blk(), blk_off(),
            blk(), blk_off(),
            blk(), blk_off(),
            blk(), blk_off(),
            wspec, wspec, wspec, wspec, wspec,
            vspec, vspec, vspec, vspec, vspec,
        ],
        out_specs=pl.BlockSpec((NB_BLK, D), lambda i: (i, 0)),
        out_shape=jax.ShapeDtypeStruct((N, D), jnp.float32),
    )(x, summ, summ, sumw, sumw, cntm, cntm, cntw, cntw,
      W1a, W1b, W1c, W2, W3, b1, b2, b3, g, beta)


def _pad2d(idx, rows):
    return jnp.pad(idx, (0, rows * CH - idx.shape[0])).reshape(rows, CH)


def kernel(x, mesh_edge_index, mesh_edge_attr, world_edge_index, world_edge_attr,
           W1, b1, W2, b2, W3, b3, g, beta):
    steps = W1.shape[0]
    sm, rm = mesh_edge_index[0], mesh_edge_index[1]
    sw, rw = world_edge_index[0], world_edge_index[1]

    idx_m0 = _pad2d(jnp.concatenate([sm[:EMH], rm[:EMH]]), GM // 2)
    idx_m1 = _pad2d(jnp.concatenate([sm[EMH:], rm[EMH:]]), GM // 2)
    idx_gw = _pad2d(jnp.concatenate([sw, rw]), GW)
    rm2 = _pad2d(rm, MC + MBUF)
    rw2 = _pad2d(rw, WC + WBUF)
    zt = jnp.zeros((N // 16, D), jnp.float32)
    ones_t = jnp.ones((CH, D), jnp.float32)

    cntm = _counts_mesh(ones_t, rm2, zt)
    cntw = _counts_world(ones_t, rw2, zt)

    def wp(si, m):
        return (W1[si, m, :D], W1[si, m, D:2 * D], W1[si, m, 2 * D:],
                W2[si, m], W3[si, m],
                b1[si, m].reshape(1, D), b2[si, m].reshape(1, D),
                b3[si, m].reshape(1, D), g[si, m].reshape(1, D),
                beta[si, m].reshape(1, D))

    for si in range(steps):
        g0 = _gather_mesh_h(x, idx_m0)
        g1 = _gather_mesh_h(x, idx_m1)
        gat_w = _gather_world(x, idx_gw)
        half0 = _edge_mesh_h0(g0, mesh_edge_attr, *wp(si, 0))
        mesh_edge_attr = _edge_mesh_h1(g1, mesh_edge_attr, half0, *wp(si, 0))
        summ = _scatter_mesh(mesh_edge_attr, rm2, zt)
        world_edge_attr = _edge_world(gat_w, world_edge_attr, *wp(si, 1))
        sumw = _scatter_world(world_edge_attr, rw2, zt)
        x = _node(x, summ, sumw, cntm, cntw, *wp(si, 2))

    return (x, mesh_edge_attr, world_edge_attr)
